# Initial kernel scaffold; baseline (speedup 1.0000x reference)
import jax
import jax.numpy as jnp
from jax.experimental import pallas as pl


def kernel(inputs):
    raise NotImplementedError("write your pallas kernel here")


# speedup vs baseline: 1.9187x; 1.9187x over previous
import jax
import jax.numpy as jnp
from jax.experimental import pallas as pl


def _rev_body(x_ref, j_ref, o_ref):
    o_ref[...] = jnp.dot(x_ref[...], j_ref[...], preferred_element_type=jnp.float32)


def kernel(inputs):
    B, S, F = inputs.shape
    x = inputs.reshape(B * S, F)
    ROWS = 1024
    LANES = 128
    nlane_blocks = F // LANES
    flip128 = jnp.flip(jnp.eye(LANES, dtype=jnp.float32), axis=1)
    out = pl.pallas_call(
        _rev_body,
        grid=(B * S // ROWS, nlane_blocks),
        in_specs=[
            pl.BlockSpec((ROWS, LANES), lambda i, j, nb=nlane_blocks: (i, nb - 1 - j)),
            pl.BlockSpec((LANES, LANES), lambda i, j: (0, 0)),
        ],
        out_specs=pl.BlockSpec((ROWS, LANES), lambda i, j: (i, j)),
        out_shape=jax.ShapeDtypeStruct((B * S, F), x.dtype),
    )(x, flip128)
    return out.reshape(B, S, F)


# speedup vs baseline: 1.9427x; 1.0125x over previous
import jax
import jax.numpy as jnp
from jax.experimental import pallas as pl


def _rev_body(x_ref, o_ref):
    rows, lanes = x_ref.shape
    idx = jax.lax.broadcasted_iota(jnp.int32, (rows, lanes), 1)
    o_ref[...] = jnp.take_along_axis(
        x_ref[...], lanes - 1 - idx, axis=1, mode="promise_in_bounds"
    )


def kernel(inputs):
    B, S, F = inputs.shape
    x = inputs.reshape(B * S, F)
    ROWS = 1024
    LANES = 128
    nlane_blocks = F // LANES
    out = pl.pallas_call(
        _rev_body,
        grid=(B * S // ROWS, nlane_blocks),
        in_specs=[
            pl.BlockSpec((ROWS, LANES), lambda i, j, nb=nlane_blocks: (i, nb - 1 - j)),
        ],
        out_specs=pl.BlockSpec((ROWS, LANES), lambda i, j: (i, j)),
        out_shape=jax.ShapeDtypeStruct((B * S, F), x.dtype),
    )(x)
    return out.reshape(B, S, F)
